# baseline (speedup 1.0000x reference)
import jax
import jax.numpy as jnp
from jax import lax
from jax.experimental import pallas as pl
from jax.experimental.pallas import tpu as pltpu
from jax.experimental.pallas import tpu_sc as plsc

B = 4
C = 16
H = 512
W = 512
HP = H + 3
NPTS = H * W
CLIP_HI = 513.0

NC = 2
NS = 16
BATCHES_PER_CORE = B // NC
PTS_PER_TILE = NPTS // NS
CHUNK = 2048
NCHUNK = PTS_PER_TILE // CHUNK
IDXROW = 128
NSTREAM = CHUNK * 4 // IDXROW
RPAD = 266240
ZWORDS = RPAD * BATCHES_PER_CORE // NS
ZBUF = 16640
OUT_ROWS_PER_TILE = H // NS
RB = 8
ROWBUF = 4128


def _sc_body(gi_hbm, gj_hbm, out_hbm, gibuf, gjbuf, idxbuf, valbuf, zbuf,
             rowbuf, bbuf, acc, sem_in, sem_sc, sem_out):
    cid = lax.axis_index("c")
    sid = lax.axis_index("s")

    zvec = jnp.zeros((16,), jnp.float32)

    def zinit(v, carry):
        zbuf[pl.ds(v * 16, 16)] = zvec
        return carry

    lax.fori_loop(0, ZBUF // 16, zinit, 0)
    for z in range(ZWORDS // ZBUF):
        pltpu.sync_copy(zbuf, acc.at[pl.ds(sid * ZWORDS + z * ZBUF, ZBUF)])
    plsc.subcore_barrier()

    NTOT = BATCHES_PER_CORE * NCHUNK
    IN_BYTES = 2 * CHUNK * 4
    SC_BYTES = NSTREAM * IDXROW * 4

    def fire_inputs(ch):
        p = ch // NCHUNK
        batch = BATCHES_PER_CORE * cid + p
        off = batch * NPTS + sid * PTS_PER_TILE + (ch % NCHUNK) * CHUNK
        s = ch % 2
        pltpu.async_copy(gi_hbm.at[pl.ds(off, CHUNK)], gibuf.at[s], sem_in)
        pltpu.async_copy(gj_hbm.at[pl.ds(off, CHUNK)], gjbuf.at[s], sem_in)

    def compute_chunk(ch):
        s = ch % 2
        plane = (ch // NCHUNK) * RPAD

        def vec_body(v, carry2):
            gi = gibuf[s, pl.ds(v * 16, 16)]
            gj = gjbuf[s, pl.ds(v * 16, 16)]
            fi = gi * (0.5 * H) + (0.5 * H + 1.0)
            fj = gj * (0.5 * W) + (0.5 * W + 1.0)
            fi = jnp.minimum(jnp.maximum(fi, 0.0), CLIP_HI)
            fj = jnp.minimum(jnp.maximum(fj, 0.0), CLIP_HI)
            li = fi.astype(jnp.int32)
            lj = fj.astype(jnp.int32)
            wi1 = fi - li.astype(jnp.float32)
            wj1 = fj - lj.astype(jnp.float32)
            wi0 = 1.0 - wi1
            wj0 = 1.0 - wj1
            e00 = plane + li * HP + lj
            row = v // (IDXROW // 16)
            col = (v * 16) % IDXROW
            nseg = CHUNK // IDXROW
            idxbuf[s, 0 * nseg + row, pl.ds(col, 16)] = e00
            idxbuf[s, 1 * nseg + row, pl.ds(col, 16)] = e00 + 1
            idxbuf[s, 2 * nseg + row, pl.ds(col, 16)] = e00 + HP
            idxbuf[s, 3 * nseg + row, pl.ds(col, 16)] = e00 + HP + 1
            vb = v * 16
            valbuf[s, pl.ds(0 * CHUNK + vb, 16)] = wi0 * wj0
            valbuf[s, pl.ds(1 * CHUNK + vb, 16)] = wi0 * wj1
            valbuf[s, pl.ds(2 * CHUNK + vb, 16)] = wi1 * wj0
            valbuf[s, pl.ds(3 * CHUNK + vb, 16)] = wi1 * wj1
            return carry2

        lax.fori_loop(0, CHUNK // 16, vec_body, 0, unroll=8)

    def fire_scatter(ch):
        s = ch % 2

        def stream_body(j, carry):
            pltpu.async_copy(valbuf.at[s, pl.ds(j * IDXROW, IDXROW)],
                             acc.at[idxbuf.at[s, j]], sem_sc, add=True)
            return carry

        lax.fori_loop(0, NSTREAM, stream_body, 0, unroll=8)

    fire_inputs(0)

    def drain_in(s):
        pltpu.make_async_copy(gi_hbm.at[pl.ds(0, CHUNK)], gibuf.at[s],
                              sem_in).wait()
        pltpu.make_async_copy(gj_hbm.at[pl.ds(0, CHUNK)], gjbuf.at[s],
                              sem_in).wait()

    def drain_sc(s):
        pltpu.make_async_copy(gi_hbm.at[pl.ds(0, CHUNK * 4)], valbuf.at[s],
                              sem_sc).wait()

    def chunk_body(ch, carry):
        s = ch % 2

        @pl.when(ch + 1 < NTOT)
        def _():
            fire_inputs(ch + 1)

        drain_in(s)

        @pl.when(ch >= 2)
        def _():
            drain_sc(s)

        compute_chunk(ch)
        fire_scatter(ch)
        return carry

    lax.fori_loop(0, NTOT, chunk_body, 0)
    drain_sc(0)
    drain_sc(1)
    plsc.subcore_barrier()

    def drain_out(s):
        pltpu.make_async_copy(gi_hbm.at[pl.ds(0, RB * W)], bbuf.at[s],
                              sem_out).wait()

    for p in range(BATCHES_PER_CORE):
        batch = BATCHES_PER_CORE * cid + p

        def block_body(blk, carry):
            io0 = sid * OUT_ROWS_PER_TILE + blk * RB
            base0 = p * RPAD + (io0 + 1) * HP + 1
            abase = (base0 // 8) * 8
            delta = base0 - abase
            s = blk % 2
            pltpu.sync_copy(acc.at[pl.ds(abase, ROWBUF)], rowbuf)

            @pl.when(blk >= 2)
            def _():
                for _c in range(C):
                    drain_out(s)

            def copy_body(t, carry2):
                d = t // (W // 16)
                j = t % (W // 16)
                vec = rowbuf[pl.ds(delta + d * HP + j * 16, 16)]
                bbuf[s, pl.ds(d * W + j * 16, 16)] = vec
                return carry2

            lax.fori_loop(0, RB * (W // 16), copy_body, 0, unroll=8)
            for c in range(C):
                pltpu.async_copy(
                    bbuf.at[s],
                    out_hbm.at[pl.ds(((batch * C + c) * H + io0) * W, RB * W)],
                    sem_out)
            return carry

        lax.fori_loop(0, OUT_ROWS_PER_TILE // RB, block_body, 0)
        for _ in range(2 * C):
            drain_out(0)


@jax.jit
def _sc_scatter(gi_flat, gj_flat):
    mesh = plsc.VectorSubcoreMesh(core_axis_name="c", subcore_axis_name="s")
    return pl.kernel(
        _sc_body,
        out_type=jax.ShapeDtypeStruct((B * C * H * W,), jnp.float32),
        mesh=mesh,
        scratch_types=[
            pltpu.VMEM((2, CHUNK), jnp.float32),
            pltpu.VMEM((2, CHUNK), jnp.float32),
            pltpu.VMEM((2, CHUNK * 4 // IDXROW, IDXROW), jnp.int32),
            pltpu.VMEM((2, CHUNK * 4), jnp.float32),
            pltpu.VMEM((ZBUF,), jnp.float32),
            pltpu.VMEM((ROWBUF,), jnp.float32),
            pltpu.VMEM((2, RB * W), jnp.float32),
            pltpu.VMEM_SHARED((RPAD * BATCHES_PER_CORE,), jnp.float32),
            pltpu.SemaphoreType.DMA,
            pltpu.SemaphoreType.DMA,
            pltpu.SemaphoreType.DMA,
        ],
    )(gi_flat, gj_flat)


BCROWS = 128


def _bcast_body(b_ref, o_ref):
    o_ref[...] = jnp.broadcast_to(b_ref[...][:, None, :, :],
                                  (1, C, BCROWS, W))


@jax.jit
def _broadcast(b3):
    return pl.pallas_call(
        _bcast_body,
        grid=(B, H // BCROWS),
        in_specs=[pl.BlockSpec((1, BCROWS, W), lambda b, r: (b, r, 0))],
        out_specs=pl.BlockSpec((1, C, BCROWS, W), lambda b, r: (b, 0, r, 0)),
        out_shape=jax.ShapeDtypeStruct((B, C, H, W), jnp.float32),
    )(b3)


def kernel(x, inv_grid):
    del x
    gi = inv_grid[..., 0].reshape(-1)
    gj = inv_grid[..., 1].reshape(-1)
    return _sc_scatter(gi, gj).reshape(B, C, H, W)


# speedup vs baseline: 1.3266x; 1.3266x over previous
import jax
import jax.numpy as jnp
from jax import lax
from jax.experimental import pallas as pl
from jax.experimental.pallas import tpu as pltpu
from jax.experimental.pallas import tpu_sc as plsc

B = 4
C = 16
H = 512
W = 512
HP = H + 3
NPTS = H * W
CLIP_HI = 513.0

NC = 2
NS = 16
BATCHES_PER_CORE = B // NC
PTS_PER_TILE = NPTS // NS
CHUNK = 2048
NCHUNK = PTS_PER_TILE // CHUNK
IDXROW = 128
NSTREAM = CHUNK * 4 // IDXROW
RPAD = 266240
ZWORDS = RPAD * BATCHES_PER_CORE // NS
ZBUF = 16640
OUT_ROWS_PER_TILE = H // NS
RB = 8
ROWBUF = 4128


def _sc_body(g2_hbm, b3_hbm, gibuf, gjbuf, idxbuf, valbuf, zbuf,
             rowbuf, bbuf, acc, sem_in, sem_sc):
    cid = lax.axis_index("c")
    sid = lax.axis_index("s")

    zvec = jnp.zeros((16,), jnp.float32)

    def zinit(v, carry):
        zbuf[pl.ds(v * 16, 16)] = zvec
        return carry

    lax.fori_loop(0, ZBUF // 16, zinit, 0, unroll=8)
    for z in range(ZWORDS // ZBUF):
        pltpu.sync_copy(zbuf, acc.at[pl.ds(sid * ZWORDS + z * ZBUF, ZBUF)])
    plsc.subcore_barrier()

    NTOT = BATCHES_PER_CORE * NCHUNK
    IN_BYTES = 2 * CHUNK * 4
    SC_BYTES = NSTREAM * IDXROW * 4

    def fire_inputs(ch):
        p = ch // NCHUNK
        batch = BATCHES_PER_CORE * cid + p
        off = batch * NPTS + sid * PTS_PER_TILE + (ch % NCHUNK) * CHUNK
        s = ch % 2
        pltpu.async_copy(g2_hbm.at[pl.ds(off, CHUNK)], gibuf.at[s], sem_in)
        pltpu.async_copy(g2_hbm.at[pl.ds(B * NPTS + off, CHUNK)],
                         gjbuf.at[s], sem_in)

    def compute_chunk(ch):
        s = ch % 2
        plane = (ch // NCHUNK) * RPAD

        def vec_body(v, carry2):
            gi = gibuf[s, pl.ds(v * 16, 16)]
            gj = gjbuf[s, pl.ds(v * 16, 16)]
            fi = gi * (0.5 * H) + (0.5 * H + 1.0)
            fj = gj * (0.5 * W) + (0.5 * W + 1.0)
            fi = jnp.minimum(jnp.maximum(fi, 0.0), CLIP_HI)
            fj = jnp.minimum(jnp.maximum(fj, 0.0), CLIP_HI)
            li = fi.astype(jnp.int32)
            lj = fj.astype(jnp.int32)
            wi1 = fi - li.astype(jnp.float32)
            wj1 = fj - lj.astype(jnp.float32)
            wi0 = 1.0 - wi1
            wj0 = 1.0 - wj1
            e00 = plane + li * HP + lj
            row = v // (IDXROW // 16)
            col = (v * 16) % IDXROW
            nseg = CHUNK // IDXROW
            idxbuf[s, 0 * nseg + row, pl.ds(col, 16)] = e00
            idxbuf[s, 1 * nseg + row, pl.ds(col, 16)] = e00 + 1
            idxbuf[s, 2 * nseg + row, pl.ds(col, 16)] = e00 + HP
            idxbuf[s, 3 * nseg + row, pl.ds(col, 16)] = e00 + HP + 1
            vb = v * 16
            valbuf[s, pl.ds(0 * CHUNK + vb, 16)] = wi0 * wj0
            valbuf[s, pl.ds(1 * CHUNK + vb, 16)] = wi0 * wj1
            valbuf[s, pl.ds(2 * CHUNK + vb, 16)] = wi1 * wj0
            valbuf[s, pl.ds(3 * CHUNK + vb, 16)] = wi1 * wj1
            return carry2

        lax.fori_loop(0, CHUNK // 16, vec_body, 0, unroll=8)

    def fire_scatter(ch):
        s = ch % 2

        def stream_body(j, carry):
            pltpu.async_copy(valbuf.at[s, pl.ds(j * IDXROW, IDXROW)],
                             acc.at[idxbuf.at[s, j]], sem_sc, add=True)
            return carry

        lax.fori_loop(0, NSTREAM, stream_body, 0, unroll=8)

    fire_inputs(0)

    def drain_in(s):
        pltpu.make_async_copy(g2_hbm.at[pl.ds(0, CHUNK)], gibuf.at[s],
                              sem_in).wait()
        pltpu.make_async_copy(g2_hbm.at[pl.ds(0, CHUNK)], gjbuf.at[s],
                              sem_in).wait()

    def drain_sc(s):
        pltpu.make_async_copy(g2_hbm.at[pl.ds(0, CHUNK * 4)], valbuf.at[s],
                              sem_sc).wait()

    def chunk_body(ch, carry):
        s = ch % 2

        @pl.when(ch + 1 < NTOT)
        def _():
            fire_inputs(ch + 1)

        drain_in(s)

        @pl.when(ch >= 2)
        def _():
            drain_sc(s)

        compute_chunk(ch)
        fire_scatter(ch)
        return carry

    lax.fori_loop(0, NTOT, chunk_body, 0)
    drain_sc(0)
    drain_sc(1)
    plsc.subcore_barrier()

    for p in range(BATCHES_PER_CORE):
        batch = BATCHES_PER_CORE * cid + p

        def block_body(blk, carry):
            io0 = sid * OUT_ROWS_PER_TILE + blk * RB
            base0 = p * RPAD + (io0 + 1) * HP + 1
            abase = (base0 // 8) * 8
            delta = base0 - abase
            pltpu.sync_copy(acc.at[pl.ds(abase, ROWBUF)], rowbuf)

            def copy_body(t, carry2):
                d = t // (W // 16)
                j = t % (W // 16)
                vec = rowbuf[pl.ds(delta + d * HP + j * 16, 16)]
                bbuf[pl.ds(d * W + j * 16, 16)] = vec
                return carry2

            lax.fori_loop(0, RB * (W // 16), copy_body, 0, unroll=8)
            pltpu.sync_copy(
                bbuf, b3_hbm.at[pl.ds((batch * H + io0) * W, RB * W)])
            return carry

        lax.fori_loop(0, OUT_ROWS_PER_TILE // RB, block_body, 0)


@jax.jit
def _sc_scatter(g2_flat):
    mesh = plsc.VectorSubcoreMesh(core_axis_name="c", subcore_axis_name="s")
    return pl.kernel(
        _sc_body,
        out_type=jax.ShapeDtypeStruct((B * H * W,), jnp.float32),
        mesh=mesh,
        scratch_types=[
            pltpu.VMEM((2, CHUNK), jnp.float32),
            pltpu.VMEM((2, CHUNK), jnp.float32),
            pltpu.VMEM((2, CHUNK * 4 // IDXROW, IDXROW), jnp.int32),
            pltpu.VMEM((2, CHUNK * 4), jnp.float32),
            pltpu.VMEM((ZBUF,), jnp.float32),
            pltpu.VMEM((ROWBUF,), jnp.float32),
            pltpu.VMEM((RB * W,), jnp.float32),
            pltpu.VMEM_SHARED((RPAD * BATCHES_PER_CORE,), jnp.float32),
            pltpu.SemaphoreType.DMA,
            pltpu.SemaphoreType.DMA,
        ],
    )(g2_flat)


BCROWS = 128


def _bcast_body(b_ref, o_ref):
    o_ref[...] = jnp.broadcast_to(b_ref[...][:, None, :, :],
                                  (1, C, BCROWS, W))


@jax.jit
def _broadcast(b3):
    return pl.pallas_call(
        _bcast_body,
        grid=(B, H // BCROWS),
        in_specs=[pl.BlockSpec((1, BCROWS, W), lambda b, r: (b, r, 0))],
        out_specs=pl.BlockSpec((1, C, BCROWS, W), lambda b, r: (b, 0, r, 0)),
        out_shape=jax.ShapeDtypeStruct((B, C, H, W), jnp.float32),
    )(b3)


def kernel(x, inv_grid):
    del x
    g2 = jnp.moveaxis(inv_grid, -1, 0).reshape(-1)
    b3 = _sc_scatter(g2).reshape(B, H, W)
    return _broadcast(b3)


# speedup vs baseline: 1.4644x; 1.1039x over previous
import jax
import jax.numpy as jnp
from jax import lax
from jax.experimental import pallas as pl
from jax.experimental.pallas import tpu as pltpu
from jax.experimental.pallas import tpu_sc as plsc

B = 4
C = 16
H = 512
W = 512
HP = H + 3
NPTS = H * W
CLIP_HI = 513.0

NC = 2
NS = 16
BATCHES_PER_CORE = B // NC
PTS_PER_TILE = NPTS // NS
CHUNK = 2048
NCHUNK = PTS_PER_TILE // CHUNK
IDXROW = 128
NSTREAM = CHUNK * 4 // IDXROW
RPAD = 266240
ZWORDS = RPAD * BATCHES_PER_CORE // NS
ZBUF = 16640
OUT_ROWS_PER_TILE = H // NS
RB = 8
ROWBUF = 4128


def _sc_body(gi_hbm, gj_hbm, b3_hbm, gibuf, gjbuf, idxbuf, valbuf, zbuf,
             rowbuf, bbuf, acc, sem_in, sem_sc):
    cid = lax.axis_index("c")
    sid = lax.axis_index("s")

    zvec = jnp.zeros((16,), jnp.float32)

    def zinit(v, carry):
        zbuf[pl.ds(v * 16, 16)] = zvec
        return carry

    lax.fori_loop(0, ZBUF // 16, zinit, 0, unroll=8)
    for z in range(ZWORDS // ZBUF):
        pltpu.sync_copy(zbuf, acc.at[pl.ds(sid * ZWORDS + z * ZBUF, ZBUF)])
    plsc.subcore_barrier()

    NTOT = BATCHES_PER_CORE * NCHUNK
    IN_BYTES = 2 * CHUNK * 4
    SC_BYTES = NSTREAM * IDXROW * 4

    def fire_inputs(ch):
        p = ch // NCHUNK
        batch = BATCHES_PER_CORE * cid + p
        off = batch * NPTS + sid * PTS_PER_TILE + (ch % NCHUNK) * CHUNK
        s = ch % 2
        pltpu.async_copy(gi_hbm.at[pl.ds(off, CHUNK)], gibuf.at[s], sem_in)
        pltpu.async_copy(gj_hbm.at[pl.ds(off, CHUNK)], gjbuf.at[s], sem_in)

    def compute_chunk(ch):
        s = ch % 2
        plane = (ch // NCHUNK) * RPAD

        def vec_body(v, carry2):
            gi = gibuf[s, pl.ds(v * 16, 16)]
            gj = gjbuf[s, pl.ds(v * 16, 16)]
            fi = gi * (0.5 * H) + (0.5 * H + 1.0)
            fj = gj * (0.5 * W) + (0.5 * W + 1.0)
            fi = jnp.minimum(jnp.maximum(fi, 0.0), CLIP_HI)
            fj = jnp.minimum(jnp.maximum(fj, 0.0), CLIP_HI)
            li = fi.astype(jnp.int32)
            lj = fj.astype(jnp.int32)
            wi1 = fi - li.astype(jnp.float32)
            wj1 = fj - lj.astype(jnp.float32)
            wi0 = 1.0 - wi1
            wj0 = 1.0 - wj1
            e00 = plane + li * HP + lj
            row = v // (IDXROW // 16)
            col = (v * 16) % IDXROW
            nseg = CHUNK // IDXROW
            idxbuf[s, 0 * nseg + row, pl.ds(col, 16)] = e00
            idxbuf[s, 1 * nseg + row, pl.ds(col, 16)] = e00 + 1
            idxbuf[s, 2 * nseg + row, pl.ds(col, 16)] = e00 + HP
            idxbuf[s, 3 * nseg + row, pl.ds(col, 16)] = e00 + HP + 1
            vb = v * 16
            valbuf[s, pl.ds(0 * CHUNK + vb, 16)] = wi0 * wj0
            valbuf[s, pl.ds(1 * CHUNK + vb, 16)] = wi0 * wj1
            valbuf[s, pl.ds(2 * CHUNK + vb, 16)] = wi1 * wj0
            valbuf[s, pl.ds(3 * CHUNK + vb, 16)] = wi1 * wj1
            return carry2

        lax.fori_loop(0, CHUNK // 16, vec_body, 0, unroll=8)

    def fire_scatter(ch):
        s = ch % 2

        def stream_body(j, carry):
            pltpu.async_copy(valbuf.at[s, pl.ds(j * IDXROW, IDXROW)],
                             acc.at[idxbuf.at[s, j]], sem_sc, add=True)
            return carry

        lax.fori_loop(0, NSTREAM, stream_body, 0, unroll=8)

    fire_inputs(0)

    def drain_in(s):
        pltpu.make_async_copy(gi_hbm.at[pl.ds(0, CHUNK)], gibuf.at[s],
                              sem_in).wait()
        pltpu.make_async_copy(gj_hbm.at[pl.ds(0, CHUNK)], gjbuf.at[s],
                              sem_in).wait()

    def drain_sc(s):
        pltpu.make_async_copy(gi_hbm.at[pl.ds(0, CHUNK * 4)], valbuf.at[s],
                              sem_sc).wait()

    def chunk_body(ch, carry):
        s = ch % 2

        @pl.when(ch + 1 < NTOT)
        def _():
            fire_inputs(ch + 1)

        drain_in(s)

        @pl.when(ch >= 2)
        def _():
            drain_sc(s)

        compute_chunk(ch)
        fire_scatter(ch)
        return carry

    lax.fori_loop(0, NTOT, chunk_body, 0)
    drain_sc(0)
    drain_sc(1)
    plsc.subcore_barrier()

    for p in range(BATCHES_PER_CORE):
        batch = BATCHES_PER_CORE * cid + p

        def block_body(blk, carry):
            io0 = sid * OUT_ROWS_PER_TILE + blk * RB
            base0 = p * RPAD + (io0 + 1) * HP + 1
            abase = (base0 // 8) * 8
            delta = base0 - abase
            pltpu.sync_copy(acc.at[pl.ds(abase, ROWBUF)], rowbuf)

            def copy_body(t, carry2):
                d = t // (W // 16)
                j = t % (W // 16)
                vec = rowbuf[pl.ds(delta + d * HP + j * 16, 16)]
                bbuf[pl.ds(d * W + j * 16, 16)] = vec
                return carry2

            lax.fori_loop(0, RB * (W // 16), copy_body, 0, unroll=8)
            pltpu.sync_copy(
                bbuf, b3_hbm.at[pl.ds((batch * H + io0) * W, RB * W)])
            return carry

        lax.fori_loop(0, OUT_ROWS_PER_TILE // RB, block_body, 0)


@jax.jit
def _sc_scatter(gi_flat, gj_flat):
    mesh = plsc.VectorSubcoreMesh(core_axis_name="c", subcore_axis_name="s")
    return pl.kernel(
        _sc_body,
        out_type=jax.ShapeDtypeStruct((B * H * W,), jnp.float32),
        mesh=mesh,
        scratch_types=[
            pltpu.VMEM((2, CHUNK), jnp.float32),
            pltpu.VMEM((2, CHUNK), jnp.float32),
            pltpu.VMEM((2, CHUNK * 4 // IDXROW, IDXROW), jnp.int32),
            pltpu.VMEM((2, CHUNK * 4), jnp.float32),
            pltpu.VMEM((ZBUF,), jnp.float32),
            pltpu.VMEM((ROWBUF,), jnp.float32),
            pltpu.VMEM((RB * W,), jnp.float32),
            pltpu.VMEM_SHARED((RPAD * BATCHES_PER_CORE,), jnp.float32),
            pltpu.SemaphoreType.DMA,
            pltpu.SemaphoreType.DMA,
        ],
    )(gi_flat, gj_flat)


BCROWS = 128


def _bcast_body(b_ref, o_ref):
    o_ref[...] = jnp.broadcast_to(b_ref[...][:, None, :, :],
                                  (1, C, BCROWS, W))


@jax.jit
def _broadcast(b3):
    return pl.pallas_call(
        _bcast_body,
        grid=(B, H // BCROWS),
        in_specs=[pl.BlockSpec((1, BCROWS, W), lambda b, r: (b, r, 0))],
        out_specs=pl.BlockSpec((1, C, BCROWS, W), lambda b, r: (b, 0, r, 0)),
        out_shape=jax.ShapeDtypeStruct((B, C, H, W), jnp.float32),
    )(b3)


def kernel(x, inv_grid):
    del x
    gi = inv_grid[..., 0].reshape(-1)
    gj = inv_grid[..., 1].reshape(-1)
    b3 = _sc_scatter(gi, gj).reshape(B, H, W)
    return _broadcast(b3)
